# baseline (speedup 1.0000x reference)
import jax
import jax.numpy as jnp
from jax import lax
from jax.experimental import pallas as pl
from jax.experimental.pallas import tpu as pltpu
from jax.experimental.pallas import tpu_sc as plsc

_NC = 2
_NS = 16
_NW = _NC * _NS

_N = 4 * 8192
_D = 1024
_PER_W = _N // _NW
_NBUF = 4
_C = 16
_NCHUNK = _PER_W // _C
_H = _NCHUNK // _NBUF


def _gather_body(idx_hbm, table_hbm, out_hbm, idx_v, *bufs):
    rows = bufs[:_NBUF]
    gsem = bufs[_NBUF:2 * _NBUF]
    ssem = bufs[2 * _NBUF:]
    wid = lax.axis_index("s") * _NC + lax.axis_index("c")
    base = wid * _PER_W
    pltpu.sync_copy(idx_hbm.at[wid], idx_v)
    for b in range(_NBUF):
        pltpu.async_copy(table_hbm.at[idx_v.at[b]], rows[b], gsem[b])

    def group(i, carry):
        c0 = i * _NBUF
        for b in range(_NBUF):
            pltpu.make_async_copy(table_hbm.at[pl.ds(0, _C)], rows[b], gsem[b]).wait()
            pltpu.async_copy(rows[b], out_hbm.at[pl.ds(base + (c0 + b) * _C, _C)], ssem[b])

        @pl.when(i < _H - 1)
        def _():
            for b in range(_NBUF):
                pltpu.make_async_copy(rows[b], out_hbm.at[pl.ds(0, _C)], ssem[b]).wait()
                pltpu.async_copy(table_hbm.at[idx_v.at[c0 + _NBUF + b]], rows[b], gsem[b])

        return carry

    lax.fori_loop(0, _H, group, 0)
    for b in range(_NBUF):
        pltpu.make_async_copy(rows[b], out_hbm.at[pl.ds(0, _C)], ssem[b]).wait()


@jax.jit
def _sc_gather(idx, table):
    mesh = plsc.VectorSubcoreMesh(core_axis_name="c", subcore_axis_name="s")
    return pl.kernel(
        _gather_body,
        out_type=jax.ShapeDtypeStruct((_N, _D), jnp.float32),
        mesh=mesh,
        scratch_types=(
            [pltpu.VMEM((_NCHUNK, _C), jnp.int32)]
            + [pltpu.VMEM((_C, _D), jnp.float32)] * _NBUF
            + [pltpu.SemaphoreType.DMA] * (2 * _NBUF)
        ),
    )(idx, table)


def kernel(position_ids, table):
    idx = position_ids.astype(jnp.int32).reshape(_NW, _NCHUNK, _C)
    out = _sc_gather(idx, table)
    return out.reshape(position_ids.shape + (table.shape[1],))
